# Initial kernel scaffold; baseline (speedup 1.0000x reference)
import jax
import jax.numpy as jnp
from jax.experimental import pallas as pl


def kernel(class_objects, states_objects, edge_tuples, edge_classes, mask_object, mask_edge, emb, W0, C0, W1, C1, W2, C2):
    raise NotImplementedError("write your pallas kernel here")


# speedup vs baseline: 7.4477x; 7.4477x over previous
import functools

import jax
import jax.numpy as jnp
from jax import lax
from jax.experimental import pallas as pl
from jax.experimental.pallas import tpu as pltpu
from jax.experimental.pallas import tpu_sc as plsc

B = 100
NMAX = 100
EPG = 3200
HDIM = 128
ODIM = 128
NREL = 16
NBASE = 8
NCLS = 1000

SPAD = 128
KDIM = NREL * SPAD
NHALF = 50
NTASK = 2 * B
NW = 32
AWORDS = NHALF * KDIM


def _sc_body(src_hbm, dst_hbm, rel_hbm, cls_hbm, emb_hbm,
             a_hbm, h_hbm,
             a_v, src_v, dst_v, rel_v, idx_v, rows_v, sem):
  w = lax.axis_index("s") * 2 + lax.axis_index("c")
  lanes = lax.iota(jnp.int32, 16)
  zeros16 = jnp.zeros((16,), jnp.float32)
  ones16 = jnp.ones((16,), jnp.float32)

  for t in range((NTASK + NW - 1) // NW):
    tid = t * NW + w

    @pl.when(tid < NTASK)
    def _():
      b = tid // 2
      half = tid % 2
      d0 = half * NHALF

      @plsc.parallel_loop(0, AWORDS, 16, unroll=8)
      def _(i):
        a_v[pl.ds(i, 16)] = zeros16

      pltpu.sync_copy(src_hbm.at[b], src_v)
      pltpu.sync_copy(dst_hbm.at[b], dst_v)
      pltpu.sync_copy(rel_hbm.at[b], rel_v)

      def grp(g, carry):
        s = src_v[pl.ds(g * 16, 16)]
        d = dst_v[pl.ds(g * 16, 16)]
        r = rel_v[pl.ds(g * 16, 16)]
        dl = d - d0
        valid = (dl >= 0) & (dl < NHALF)
        flat = dl * KDIM + r * SPAD + s
        for lane in range(16):
          plsc.addupdate_scatter(a_v, [flat], ones16,
                                 mask=valid & (lanes == lane))
        return carry

      lax.fori_loop(0, EPG // 16, grp, 0)
      pltpu.sync_copy(a_v, a_hbm.at[b, half])

      @pl.when(half == 0)
      def _():
        pltpu.sync_copy(cls_hbm.at[b], idx_v)
        pltpu.async_copy(emb_hbm.at[idx_v.at[pl.ds(0, NMAX)]], rows_v,
                         sem).wait()
        pltpu.sync_copy(rows_v, h_hbm.at[b])


@jax.jit
def _sc_build(src, dst, rel, cls, emb):
  mesh = plsc.VectorSubcoreMesh(core_axis_name="c", subcore_axis_name="s")
  fn = pl.kernel(
      _sc_body,
      out_type=(
          jax.ShapeDtypeStruct((B, 2, AWORDS), jnp.float32),
          jax.ShapeDtypeStruct((B, NMAX, HDIM), jnp.float32),
      ),
      mesh=mesh,
      compiler_params=pltpu.CompilerParams(needs_layout_passes=False),
      scratch_types=[
          pltpu.VMEM((AWORDS,), jnp.float32),
          pltpu.VMEM((EPG,), jnp.int32),
          pltpu.VMEM((EPG,), jnp.int32),
          pltpu.VMEM((EPG,), jnp.int32),
          pltpu.VMEM((SPAD,), jnp.int32),
          pltpu.VMEM((NMAX, HDIM), jnp.float32),
          pltpu.SemaphoreType.DMA,
      ],
  )
  return fn(src, dst, rel, cls, emb)


def _tc_body(h_ref, a_ref, w0_ref, w1_ref, w2_ref, o_ref):
  zpad = jnp.zeros((SPAD - NMAX, HDIM), jnp.float32)
  hp = jnp.concatenate([h_ref[0], zpad], axis=0)
  a = a_ref[0]
  agg = None
  for w_ref, last in ((w0_ref, False), (w1_ref, False), (w2_ref, True)):
    t = jnp.dot(hp, w_ref[...], preferred_element_type=jnp.float32)
    t2 = jnp.concatenate(
        [t[:, r * 128:(r + 1) * 128] for r in range(NREL)], axis=0)
    agg = jnp.dot(a, t2, preferred_element_type=jnp.float32)
    if not last:
      hp = jnp.concatenate([jax.nn.relu(agg), zpad], axis=0)
  m = jnp.max(agg, axis=1, keepdims=True)
  e = jnp.exp(agg - m)
  o_ref[0] = e / jnp.sum(e, axis=1, keepdims=True)


def _tc_run(h, a, w0c, w1c, w2c):
  return pl.pallas_call(
      _tc_body,
      grid=(B,),
      in_specs=[
          pl.BlockSpec((1, NMAX, HDIM), lambda b: (b, 0, 0)),
          pl.BlockSpec((1, NMAX, KDIM), lambda b: (b, 0, 0)),
          pl.BlockSpec((HDIM, KDIM), lambda b: (0, 0)),
          pl.BlockSpec((HDIM, KDIM), lambda b: (0, 0)),
          pl.BlockSpec((HDIM, KDIM), lambda b: (0, 0)),
      ],
      out_specs=pl.BlockSpec((1, NMAX, ODIM), lambda b: (b, 0, 0)),
      out_shape=jax.ShapeDtypeStruct((B, NMAX, ODIM), jnp.float32),
  )(h, a, w0c, w1c, w2c)


def _mk_wcat(wb, cw):
  out_feat = wb.shape[-1]
  wv = wb.reshape(HDIM, NBASE, out_feat)
  wfull = jnp.matmul(cw, wv).reshape(NREL, HDIM, out_feat)
  return wfull.transpose(1, 0, 2).reshape(HDIM, NREL * out_feat)


def kernel(class_objects, states_objects, edge_tuples, edge_classes,
           mask_object, mask_edge, emb, W0, C0, W1, C1, W2, C2):
  src = edge_tuples[:, :, 0].astype(jnp.int32)
  dst = edge_tuples[:, :, 1].astype(jnp.int32)
  rel = edge_classes.astype(jnp.int32)
  cls = jnp.pad(class_objects.astype(jnp.int32),
                ((0, 0), (0, SPAD - NMAX)))
  a_flat, h = _sc_build(src, dst, rel, cls, emb.astype(jnp.float32))
  a = a_flat.reshape(B, NMAX, KDIM)
  w0c = _mk_wcat(W0, C0)
  w1c = _mk_wcat(W1, C1)
  w2c = _mk_wcat(W2, C2)
  return _tc_run(h, a, w0c, w1c, w2c)


# speedup vs baseline: 8.7134x; 1.1699x over previous
import functools

import jax
import jax.numpy as jnp
from jax import lax
from jax.experimental import pallas as pl
from jax.experimental.pallas import tpu as pltpu
from jax.experimental.pallas import tpu_sc as plsc

B = 100
NMAX = 100
EPG = 3200
HDIM = 128
ODIM = 128
NREL = 16
NBASE = 8
NCLS = 1000

SPAD = 128
KDIM = NREL * SPAD
NHALF = 50
NTASK = 2 * B
NW = 32
AWORDS = NHALF * KDIM


def _sc_body(src_hbm, dst_hbm, rel_hbm, cls_hbm, emb_hbm,
             a_hbm, h_hbm,
             a_v, src_v, dst_v, rel_v, idx_v, rows_v, sem):
  w = lax.axis_index("s") * 2 + lax.axis_index("c")
  lanes = lax.iota(jnp.int32, 16)
  zeros16 = jnp.zeros((16,), jnp.float32)
  ones16 = jnp.ones((16,), jnp.float32)

  for t in range((NTASK + NW - 1) // NW):
    tid = t * NW + w

    @pl.when(tid < NTASK)
    def _():
      b = tid % B
      half = tid // B
      d0 = half * NHALF

      @plsc.parallel_loop(0, AWORDS, 16, unroll=8)
      def _(i):
        a_v[pl.ds(i, 16)] = zeros16

      pltpu.sync_copy(src_hbm.at[b], src_v)
      pltpu.sync_copy(dst_hbm.at[b], dst_v)
      pltpu.sync_copy(rel_hbm.at[b], rel_v)

      def grp(g, carry):
        s = src_v[pl.ds(g * 16, 16)]
        d = dst_v[pl.ds(g * 16, 16)]
        r = rel_v[pl.ds(g * 16, 16)]
        dl = d - d0
        valid = (dl >= 0) & (dl < NHALF)
        flat = dl * KDIM + r * SPAD + s
        for lane in range(16):
          plsc.addupdate_scatter(a_v, [flat], ones16,
                                 mask=valid & (lanes == lane))
        return carry

      lax.fori_loop(0, EPG // 16, grp, 0)
      pltpu.sync_copy(a_v, a_hbm.at[b, half])

      @pl.when(half == 0)
      def _():
        pltpu.sync_copy(cls_hbm.at[b], idx_v)
        pltpu.async_copy(emb_hbm.at[idx_v.at[pl.ds(0, NMAX)]], rows_v,
                         sem).wait()
        pltpu.sync_copy(rows_v, h_hbm.at[b])


@jax.jit
def _sc_build(src, dst, rel, cls, emb):
  mesh = plsc.VectorSubcoreMesh(core_axis_name="c", subcore_axis_name="s")
  fn = pl.kernel(
      _sc_body,
      out_type=(
          jax.ShapeDtypeStruct((B, 2, AWORDS), jnp.float32),
          jax.ShapeDtypeStruct((B, NMAX, HDIM), jnp.float32),
      ),
      mesh=mesh,
      compiler_params=pltpu.CompilerParams(needs_layout_passes=False),
      scratch_types=[
          pltpu.VMEM((AWORDS,), jnp.float32),
          pltpu.VMEM((EPG,), jnp.int32),
          pltpu.VMEM((EPG,), jnp.int32),
          pltpu.VMEM((EPG,), jnp.int32),
          pltpu.VMEM((SPAD,), jnp.int32),
          pltpu.VMEM((NMAX, HDIM), jnp.float32),
          pltpu.SemaphoreType.DMA,
      ],
  )
  return fn(src, dst, rel, cls, emb)


GPP = 2


def _tc_body(h_ref, a_ref, w0_ref, w1_ref, w2_ref, o_ref):
  zpad = jnp.zeros((SPAD - NMAX, HDIM), jnp.float32)
  hp = jnp.concatenate(
      sum(([h_ref[g], zpad] for g in range(GPP)), []), axis=0)
  aggs = [None] * GPP
  for w_ref, last in ((w0_ref, False), (w1_ref, False), (w2_ref, True)):
    t = jnp.dot(hp.astype(jnp.bfloat16), w_ref[...],
                preferred_element_type=jnp.float32
                ).astype(jnp.bfloat16)
    hps = []
    for g in range(GPP):
      t2 = jnp.concatenate(
          [t[g * SPAD:(g + 1) * SPAD, r * 128:(r + 1) * 128]
           for r in range(NREL)], axis=0)
      a = a_ref[g].astype(jnp.bfloat16)
      aggs[g] = jnp.dot(a, t2, preferred_element_type=jnp.float32)
      if not last:
        hps += [jax.nn.relu(aggs[g]), zpad]
    if not last:
      hp = jnp.concatenate(hps, axis=0)
  for g in range(GPP):
    agg = aggs[g]
    m = jnp.max(agg, axis=1, keepdims=True)
    e = jnp.exp(agg - m)
    o_ref[g] = e / jnp.sum(e, axis=1, keepdims=True)


def _tc_run(h, a, w0c, w1c, w2c):
  return pl.pallas_call(
      _tc_body,
      grid=(B // GPP,),
      in_specs=[
          pl.BlockSpec((GPP, NMAX, HDIM), lambda b: (b, 0, 0)),
          pl.BlockSpec((GPP, NMAX, KDIM), lambda b: (b, 0, 0)),
          pl.BlockSpec((HDIM, KDIM), lambda b: (0, 0)),
          pl.BlockSpec((HDIM, KDIM), lambda b: (0, 0)),
          pl.BlockSpec((HDIM, KDIM), lambda b: (0, 0)),
      ],
      out_specs=pl.BlockSpec((GPP, NMAX, ODIM), lambda b: (b, 0, 0)),
      out_shape=jax.ShapeDtypeStruct((B, NMAX, ODIM), jnp.float32),
  )(h, a, w0c, w1c, w2c)


def _mk_wcat(wb, cw):
  out_feat = wb.shape[-1]
  wv = wb.reshape(HDIM, NBASE, out_feat)
  wfull = jnp.matmul(cw, wv).reshape(NREL, HDIM, out_feat)
  return wfull.transpose(1, 0, 2).reshape(HDIM, NREL * out_feat).astype(
      jnp.bfloat16)


def kernel(class_objects, states_objects, edge_tuples, edge_classes,
           mask_object, mask_edge, emb, W0, C0, W1, C1, W2, C2):
  src = edge_tuples[:, :, 0].astype(jnp.int32)
  dst = edge_tuples[:, :, 1].astype(jnp.int32)
  rel = edge_classes.astype(jnp.int32)
  cls = jnp.pad(class_objects.astype(jnp.int32),
                ((0, 0), (0, SPAD - NMAX)))
  a_flat, h = _sc_build(src, dst, rel, cls, emb.astype(jnp.float32))
  a = a_flat.reshape(B, NMAX, KDIM)
  w0c = _mk_wcat(W0, C0)
  w1c = _mk_wcat(W1, C1)
  w2c = _mk_wcat(W2, C2)
  return _tc_run(h, a, w0c, w1c, w2c)


# speedup vs baseline: 10.7788x; 1.2370x over previous
import functools

import jax
import jax.numpy as jnp
from jax import lax
from jax.experimental import pallas as pl
from jax.experimental.pallas import tpu as pltpu
from jax.experimental.pallas import tpu_sc as plsc

B = 100
NMAX = 100
EPG = 3200
HDIM = 128
ODIM = 128
NREL = 16
NBASE = 8
NCLS = 1000

SPAD = 128
KDIM = NREL * SPAD
NQ = 4
NDQ = NMAX // NQ
NTASK = NQ * B
NW = 32
QWORDS = NDQ * KDIM


def _sc_body(src_hbm, dst_hbm, rel_hbm, cls_hbm, emb_hbm,
             a_hbm, h_hbm,
             a0_v, a1_v, src_v, dst_v, rel_v, idx_v, rows_v,
             esem, gsem, osem0, osem1):
  w = lax.axis_index("s") * 2 + lax.axis_index("c")
  zeros16 = jnp.zeros((16,), jnp.float32)
  abufs = (a0_v, a1_v)
  osems = (osem0, osem1)
  pending = [None, None]

  for t in range((NTASK + NW - 1) // NW):
    tid = jnp.minimum(t * NW + w, NTASK - 1)
    p = t % 2
    b = tid % B
    q = tid // B
    d0 = q * NDQ
    a_v = abufs[p]

    e0 = pltpu.async_copy(src_hbm.at[b], src_v, esem)
    e1 = pltpu.async_copy(dst_hbm.at[b], dst_v, esem)
    e2 = pltpu.async_copy(rel_hbm.at[b], rel_v, esem)

    if pending[p] is not None:
      pending[p].wait()

    @plsc.parallel_loop(0, QWORDS, 16, unroll=8)
    def _(i):
      a_v[pl.ds(i, 16)] = zeros16

    e0.wait()
    e1.wait()
    e2.wait()

    @plsc.parallel_loop(0, EPG // 16, 1, unroll=4)
    def _(g):
      s = src_v[pl.ds(g * 16, 16)]
      d = dst_v[pl.ds(g * 16, 16)]
      r = rel_v[pl.ds(g * 16, 16)]
      dl = d - d0
      valid = (dl >= 0) & (dl < NDQ)
      flat = dl * KDIM + r * SPAD + s
      cnt, last = plsc.scan_count(flat, mask=valid)
      plsc.addupdate_scatter(a_v, [flat], cnt.astype(jnp.float32),
                             mask=last & valid)

    pending[p] = pltpu.async_copy(a_v, a_hbm.at[b, q], osems[p])

    @pl.when(q == 0)
    def _():
      pltpu.sync_copy(cls_hbm.at[b], idx_v)
      pltpu.async_copy(emb_hbm.at[idx_v.at[pl.ds(0, NMAX)]], rows_v,
                       gsem).wait()
      pltpu.sync_copy(rows_v, h_hbm.at[b])

  for d in pending:
    if d is not None:
      d.wait()


@jax.jit
def _sc_build(src, dst, rel, cls, emb):
  mesh = plsc.VectorSubcoreMesh(core_axis_name="c", subcore_axis_name="s")
  fn = pl.kernel(
      _sc_body,
      out_type=(
          jax.ShapeDtypeStruct((B, NQ, QWORDS), jnp.float32),
          jax.ShapeDtypeStruct((B, NMAX, HDIM), jnp.float32),
      ),
      mesh=mesh,
      compiler_params=pltpu.CompilerParams(needs_layout_passes=False),
      scratch_types=[
          pltpu.VMEM((QWORDS,), jnp.float32),
          pltpu.VMEM((QWORDS,), jnp.float32),
          pltpu.VMEM((EPG,), jnp.int32),
          pltpu.VMEM((EPG,), jnp.int32),
          pltpu.VMEM((EPG,), jnp.int32),
          pltpu.VMEM((SPAD,), jnp.int32),
          pltpu.VMEM((NMAX, HDIM), jnp.float32),
          pltpu.SemaphoreType.DMA,
          pltpu.SemaphoreType.DMA,
          pltpu.SemaphoreType.DMA,
          pltpu.SemaphoreType.DMA,
      ],
  )
  return fn(src, dst, rel, cls, emb)


GPP = 2


def _tc_body(h_ref, a_ref, w0_ref, w1_ref, w2_ref, o_ref):
  zpad = jnp.zeros((SPAD - NMAX, HDIM), jnp.float32)
  hp = jnp.concatenate(
      sum(([h_ref[g], zpad] for g in range(GPP)), []), axis=0)
  aggs = [None] * GPP
  for w_ref, last in ((w0_ref, False), (w1_ref, False), (w2_ref, True)):
    t = jnp.dot(hp.astype(jnp.bfloat16), w_ref[...],
                preferred_element_type=jnp.float32
                ).astype(jnp.bfloat16)
    hps = []
    for g in range(GPP):
      t2 = jnp.concatenate(
          [t[g * SPAD:(g + 1) * SPAD, r * 128:(r + 1) * 128]
           for r in range(NREL)], axis=0)
      a = a_ref[g].astype(jnp.bfloat16)
      aggs[g] = jnp.dot(a, t2, preferred_element_type=jnp.float32)
      if not last:
        hps += [jax.nn.relu(aggs[g]), zpad]
    if not last:
      hp = jnp.concatenate(hps, axis=0)
  for g in range(GPP):
    agg = aggs[g]
    m = jnp.max(agg, axis=1, keepdims=True)
    e = jnp.exp(agg - m)
    o_ref[g] = e / jnp.sum(e, axis=1, keepdims=True)


def _tc_run(h, a, w0c, w1c, w2c):
  return pl.pallas_call(
      _tc_body,
      grid=(B // GPP,),
      in_specs=[
          pl.BlockSpec((GPP, NMAX, HDIM), lambda b: (b, 0, 0)),
          pl.BlockSpec((GPP, NMAX, KDIM), lambda b: (b, 0, 0)),
          pl.BlockSpec((HDIM, KDIM), lambda b: (0, 0)),
          pl.BlockSpec((HDIM, KDIM), lambda b: (0, 0)),
          pl.BlockSpec((HDIM, KDIM), lambda b: (0, 0)),
      ],
      out_specs=pl.BlockSpec((GPP, NMAX, ODIM), lambda b: (b, 0, 0)),
      out_shape=jax.ShapeDtypeStruct((B, NMAX, ODIM), jnp.float32),
  )(h, a, w0c, w1c, w2c)


def _mk_wcat(wb, cw):
  out_feat = wb.shape[-1]
  wv = wb.reshape(HDIM, NBASE, out_feat)
  wfull = jnp.matmul(cw, wv).reshape(NREL, HDIM, out_feat)
  return wfull.transpose(1, 0, 2).reshape(HDIM, NREL * out_feat).astype(
      jnp.bfloat16)


def kernel(class_objects, states_objects, edge_tuples, edge_classes,
           mask_object, mask_edge, emb, W0, C0, W1, C1, W2, C2):
  src = edge_tuples[:, :, 0].astype(jnp.int32)
  dst = edge_tuples[:, :, 1].astype(jnp.int32)
  rel = edge_classes.astype(jnp.int32)
  cls = jnp.pad(class_objects.astype(jnp.int32),
                ((0, 0), (0, SPAD - NMAX)))
  a_flat, h = _sc_build(src, dst, rel, cls, emb.astype(jnp.float32))
  a = a_flat.reshape(B, NMAX, KDIM)
  w0c = _mk_wcat(W0, C0)
  w1c = _mk_wcat(W1, C1)
  w2c = _mk_wcat(W2, C2)
  return _tc_run(h, a, w0c, w1c, w2c)
